# baseline (speedup 1.0000x reference)
import functools

import jax
import jax.numpy as jnp
from jax import lax
from jax.experimental import pallas as pl
from jax.experimental.pallas import tpu as pltpu
from jax.experimental.pallas import tpu_sc as plsc

N = 10000
D = 128
E = 320000
G = 128
NUM_LAYERS = 5

NC, NS = 2, 16
NW = NC * NS
EPT = E // NW
HN = N // NC
BPT = 42
CH = 128
NCHA = 2 * BPT
ACC_ROWS = 5120
DUMMY = HN
ZT = 5
ZROWS = ACC_ROWS // ZT
WROWS = HN // ZT
ZCHUNK = 256

_HIGHEST = jax.lax.Precision.HIGHEST


def _dot(a, b):
    return jax.lax.dot_general(a, b, (((1,), (0,)), ((), ())),
                               preferred_element_type=jnp.float32,
                               precision=_HIGHEST)


def _dotT(a, b):
    return jax.lax.dot_general(a, b, (((0,), (0,)), ((), ())),
                               preferred_element_type=jnp.float32,
                               precision=_HIGHEST)


def _sc_mesh():
    return plsc.VectorSubcoreMesh(core_axis_name="c", subcore_axis_name="s",
                                  num_cores=NC, num_subcores=NS)


def _make_partition():
    @functools.partial(
        pl.kernel,
        mesh=_sc_mesh(),
        out_type=(jax.ShapeDtypeStruct((NC, NW, BPT, CH), jnp.int32),
                  jax.ShapeDtypeStruct((NC, NW, BPT, CH), jnp.int32)),
        scratch_types=[
            pltpu.VMEM((EPT,), jnp.int32),
            pltpu.VMEM((EPT,), jnp.int32),
            pltpu.VMEM((BPT, CH), jnp.int32),
            pltpu.VMEM((BPT, CH), jnp.int32),
            pltpu.VMEM((BPT, CH), jnp.int32),
            pltpu.VMEM((BPT, CH), jnp.int32),
        ],
        compiler_params=pltpu.CompilerParams(needs_layout_passes=False),
    )
    def partition(src_hbm, dst_hbm, bsrc_hbm, bdst_hbm,
                  sall, dall, bs0, bd0, bs1, bd1):
        cid = lax.axis_index("c")
        sid = lax.axis_index("s")
        wid = sid * NC + cid

        pltpu.sync_copy(src_hbm.at[pl.ds(wid * EPT, EPT)], sall)
        pltpu.sync_copy(dst_hbm.at[pl.ds(wid * EPT, EPT)], dall)

        zeros16 = jnp.zeros((16,), jnp.int32)
        dummy16 = jnp.full((16,), DUMMY, jnp.int32)

        def prefill(i, carry):
            for k in range(CH // 16):
                sl = pl.ds(k * 16, 16)
                bs0[i, sl] = zeros16
                bd0[i, sl] = dummy16
                bs1[i, sl] = zeros16
                bd1[i, sl] = dummy16
            return carry

        lax.fori_loop(0, BPT, prefill, 0)

        cap = BPT * CH - 1

        def classify(g, offs):
            off0, off1 = offs
            sl = pl.ds(g * 16, 16)
            s = sall[sl]
            d = dall[sl]
            m0 = d < HN
            m0i = m0.astype(jnp.int32)
            m1i = 1 - m0i
            c0 = plsc.cumsum(m0i)
            c1 = plsc.cumsum(m1i)
            n0 = lax.reduce_sum_p.bind(m0i, axes=(0,))
            pos0 = jnp.minimum(off0 + c0 - m0i, cap)
            pos1 = jnp.minimum(off1 + c1 - m1i, cap)
            r0 = lax.shift_right_logical(pos0, 7)
            l0 = lax.bitwise_and(pos0, 127)
            r1 = lax.shift_right_logical(pos1, 7)
            l1 = lax.bitwise_and(pos1, 127)
            plsc.store_scatter(bs0, [r0, l0], s, mask=m0)
            plsc.store_scatter(bd0, [r0, l0], d, mask=m0)
            plsc.store_scatter(bs1, [r1, l1], s, mask=~m0)
            plsc.store_scatter(bd1, [r1, l1], d - HN, mask=~m0)
            return off0 + n0, off1 + (16 - n0)

        lax.fori_loop(0, EPT // 16, classify,
                      (jnp.int32(0), jnp.int32(0)))

        pltpu.sync_copy(bs0, bsrc_hbm.at[0, wid])
        pltpu.sync_copy(bd0, bdst_hbm.at[0, wid])
        pltpu.sync_copy(bs1, bsrc_hbm.at[1, wid])
        pltpu.sync_copy(bd1, bdst_hbm.at[1, wid])

    return partition


def _make_edge_agg():
    @functools.partial(
        pl.kernel,
        mesh=_sc_mesh(),
        out_type=jax.ShapeDtypeStruct((N, D), jnp.float32),
        scratch_types=[
            pltpu.VMEM((NCHA, CH), jnp.int32),
            pltpu.VMEM((NCHA, CH), jnp.int32),
            pltpu.VMEM((CH, D), jnp.float32),
            pltpu.VMEM((CH, D), jnp.float32),
            pltpu.VMEM_SHARED((ACC_ROWS, D), jnp.float32),
            pltpu.SemaphoreType.DMA,
            pltpu.SemaphoreType.DMA,
        ],
    )
    def edge_agg(h_hbm, bsrc_hbm, bdst_hbm, zeros_hbm, out_hbm,
                 sidx, didx, rows0, rows1, acc, sem0, sem1):
        cid = lax.axis_index("c")
        sid = lax.axis_index("s")

        pltpu.sync_copy(bsrc_hbm.at[cid, 2 * sid], sidx.at[pl.ds(0, BPT)])
        pltpu.sync_copy(bsrc_hbm.at[cid, 2 * sid + 1],
                        sidx.at[pl.ds(BPT, BPT)])
        pltpu.sync_copy(bdst_hbm.at[cid, 2 * sid], didx.at[pl.ds(0, BPT)])
        pltpu.sync_copy(bdst_hbm.at[cid, 2 * sid + 1],
                        didx.at[pl.ds(BPT, BPT)])

        @pl.when(sid < ZT)
        def _():
            for k in range(ZROWS // ZCHUNK):
                pltpu.sync_copy(
                    zeros_hbm,
                    acc.at[pl.ds(sid * ZROWS + k * ZCHUNK, ZCHUNK)])

        plsc.subcore_barrier()

        pltpu.async_copy(h_hbm.at[sidx.at[0]], rows0, sem0)

        def body(t, carry):
            j0 = 2 * t
            j1 = 2 * t + 1
            pltpu.make_async_copy(h_hbm.at[sidx.at[j0]], rows0, sem0).wait()
            pltpu.async_copy(h_hbm.at[sidx.at[j1]], rows1, sem1)
            pltpu.sync_copy(rows0, acc.at[didx.at[j0]], add=True)
            pltpu.make_async_copy(h_hbm.at[sidx.at[j1]], rows1, sem1).wait()

            @pl.when(t < NCHA // 2 - 1)
            def _():
                pltpu.async_copy(h_hbm.at[sidx.at[j0 + 2]], rows0, sem0)

            pltpu.sync_copy(rows1, acc.at[didx.at[j1]], add=True)
            return carry

        lax.fori_loop(0, NCHA // 2, body, 0)
        plsc.subcore_barrier()

        @pl.when(sid < ZT)
        def _():
            pltpu.sync_copy(
                acc.at[pl.ds(sid * WROWS, WROWS)],
                out_hbm.at[pl.ds(cid * HN + sid * WROWS, WROWS)])

    return edge_agg


_SC_CACHE = {}


def _partition(src, dst):
    if "part" not in _SC_CACHE:
        _SC_CACHE["part"] = _make_partition()
    return _SC_CACHE["part"](src, dst)


def _edge_agg(h, bsrc, bdst, zeros):
    if "agg" not in _SC_CACHE:
        _SC_CACHE["agg"] = _make_edge_agg()
    return _SC_CACHE["agg"](h, bsrc, bdst, zeros)


_BLK = 400
_NBLK = N // _BLK

_MLP_IN_SPECS = [
    pl.BlockSpec(memory_space=pltpu.SMEM),
    pl.BlockSpec((_BLK, D), lambda i: (i, 0)),
    pl.BlockSpec((_BLK, D), lambda i: (i, 0)),
    pl.BlockSpec((D, 2 * D), lambda i: (0, 0)),
    pl.BlockSpec((1, 2 * D), lambda i: (0, 0)),
    pl.BlockSpec((2 * D, D), lambda i: (0, 0)),
    pl.BlockSpec((1, D), lambda i: (0, 0)),
]


def _mlp_common(eps_ref, h_ref, p_ref, w1_ref, b1_ref, w2_ref, b2_ref):
    m = h_ref[...] * (1.0 + eps_ref[0]) + p_ref[...]
    t = jnp.maximum(_dot(m, w1_ref[...]) + b1_ref[...], 0.0)
    return _dot(t, w2_ref[...]) + b2_ref[...]


def _mlp_layer(h, agg, W1, b1, W2, b2, eps):
    def body(eps_ref, h_ref, p_ref, w1_ref, b1_ref, w2_ref, b2_ref, o_ref):
        o = _mlp_common(eps_ref, h_ref, p_ref, w1_ref, b1_ref, w2_ref,
                        b2_ref)
        o_ref[...] = jnp.maximum(o, 0.0)

    return pl.pallas_call(
        body,
        grid=(_NBLK,),
        in_specs=_MLP_IN_SPECS,
        out_specs=pl.BlockSpec((_BLK, D), lambda i: (i, 0)),
        out_shape=jax.ShapeDtypeStruct((N, D), jnp.float32),
    )(eps, h, agg, W1, b1, W2, b2)


def _mlp_pool_layer(h, agg, W1, b1, W2, b2, eps, batch3d):
    def body(eps_ref, h_ref, p_ref, w1_ref, b1_ref, w2_ref, b2_ref,
             batch_ref, o_ref, pool_acc, cnt_acc):
        i = pl.program_id(0)
        o = _mlp_common(eps_ref, h_ref, p_ref, w1_ref, b1_ref, w2_ref,
                        b2_ref)
        seg = batch_ref[0]
        oh = (seg == lax.broadcasted_iota(jnp.int32, (_BLK, G), 1))
        oh = oh.astype(jnp.float32)
        pc = _dotT(oh, o)
        cc = _dotT(oh, jnp.ones((_BLK, D), jnp.float32))

        @pl.when(i == 0)
        def _():
            pool_acc[...] = pc
            cnt_acc[...] = cc

        @pl.when(i > 0)
        def _():
            pool_acc[...] += pc
            cnt_acc[...] += cc

        @pl.when(i == _NBLK - 1)
        def _():
            o_ref[...] = pool_acc[...] / jnp.maximum(cnt_acc[...], 1.0)

    return pl.pallas_call(
        body,
        grid=(_NBLK,),
        in_specs=_MLP_IN_SPECS + [
            pl.BlockSpec((1, _BLK, 1), lambda i: (i, 0, 0)),
        ],
        out_specs=pl.BlockSpec((G, D), lambda i: (0, 0)),
        out_shape=jax.ShapeDtypeStruct((G, D), jnp.float32),
        scratch_shapes=[
            pltpu.VMEM((G, D), jnp.float32),
            pltpu.VMEM((G, D), jnp.float32),
        ],
        compiler_params=pltpu.CompilerParams(
            dimension_semantics=("arbitrary",)),
    )(eps, h, agg, W1, b1, W2, b2, batch3d)


def _head(h1p, h2p, w1a, w1b, b1, w2p, b2p, w3p, b3p):
    def body(h1_ref, h2_ref, w1a_ref, w1b_ref, b1_ref, w2_ref, b2_ref,
             w3_ref, b3_ref, o_ref):
        z = jnp.maximum(_dot(h1_ref[...], w1a_ref[...])
                        + _dot(h2_ref[...], w1b_ref[...]) + b1_ref[...], 0.0)
        z2 = jnp.maximum(_dot(z, w2_ref[...]) + b2_ref[...], 0.0)
        o_ref[...] = _dot(z2, w3_ref[...]) + b3_ref[...]

    return pl.pallas_call(
        body,
        out_shape=jax.ShapeDtypeStruct((G, D), jnp.float32),
    )(h1p, h2p, w1a, w1b, b1, w2p, b2p, w3p, b3p)


def kernel(x1, x2, edge_index1, edge_index2, batch1, batch2,
           gin_W1, gin_b1, gin_W2, gin_b2, gin_eps,
           fc1_W, fc1_b, fc2_W, fc2_b, fc3_W, fc3_b):
    ei1 = edge_index1.astype(jnp.int32)
    ei2 = edge_index2.astype(jnp.int32)
    zeros_hbm = jnp.zeros((ZCHUNK, D), jnp.float32)
    batch1_3d = batch1.astype(jnp.int32).reshape(_NBLK, _BLK, 1)
    batch2_3d = batch2.astype(jnp.int32).reshape(_NBLK, _BLK, 1)

    b1r = gin_b1.reshape(NUM_LAYERS, 1, 2 * D)
    b2r = gin_b2.reshape(NUM_LAYERS, 1, D)

    def gin(x, ei, batch3d):
        bsrc, bdst = _partition(ei[0], ei[1])
        h = x
        for l in range(NUM_LAYERS):
            agg = _edge_agg(h, bsrc, bdst, zeros_hbm)
            eps = gin_eps[l].reshape(1)
            if l < NUM_LAYERS - 1:
                h = _mlp_layer(h, agg, gin_W1[l], b1r[l], gin_W2[l],
                               b2r[l], eps)
            else:
                h = _mlp_pool_layer(h, agg, gin_W1[l], b1r[l], gin_W2[l],
                                    b2r[l], eps, batch3d)
        return h

    h1p = gin(x1, ei1, batch1_3d)
    h2p = gin(x2, ei2, batch2_3d)

    w1a = fc1_W[:D]
    w1b = fc1_W[D:]
    b1h = fc1_b.reshape(1, D)
    w2p = jnp.pad(fc2_W, ((0, 0), (0, D - 64)))
    b2p = jnp.pad(fc2_b, (0, D - 64)).reshape(1, D)
    w3p = jnp.pad(fc3_W, ((0, D - 64), (0, D - 1)))
    b3p = jnp.pad(fc3_b, (0, D - 1)).reshape(1, D)

    out = _head(h1p, h2p, w1a, w1b, b1h, w2p, b2p, w3p, b3p)
    return out[:, :1]
